# baseline (speedup 1.0000x reference)
import jax
import jax.numpy as jnp
from jax.experimental import pallas as pl

_NEGATIVE = -2
_UPPER = 0.4
_LOWER = 0.1
_NCLS = 80
_BACKGROUND = _NCLS
_TL = 1000


def _loss_kernel(cls_ref, bd_ref, rois_ref, gtt_ref, g_ref, acc_ref):
    t = pl.program_id(1)
    cls = cls_ref[0]
    bd = bd_ref[0]
    rois = rois_ref[0]
    gtt = gtt_ref[0]
    g = g_ref[0]

    tl, C = cls.shape
    M = gtt.shape[1]
    hi = jax.lax.Precision.HIGHEST

    lane_c = jax.lax.broadcasted_iota(jnp.int32, (tl, C), 1)
    rowmax = jnp.max(cls, axis=1, keepdims=True)
    idx = jnp.min(jnp.where(cls == rowmax, lane_c, C), axis=1, keepdims=True)

    logz = jnp.log(jnp.sum(jnp.exp(cls), axis=1, keepdims=True))

    D = bd.shape[1]
    lane_d = jax.lax.broadcasted_iota(jnp.int32, (tl, D), 1)
    cls_hit = jax.lax.shift_right_logical(lane_d, 2) == idx
    q = jnp.where(cls_hit, bd, 0.0)
    tail = jnp.concatenate(
        [q[:, 256:D], jnp.zeros((tl, 384 - D), jnp.float32)], axis=1)
    qf = q[:, 0:128] + q[:, 128:256] + tail
    s_row = jax.lax.broadcasted_iota(jnp.int32, (128, 4), 0)
    s_col = jax.lax.broadcasted_iota(jnp.int32, (128, 4), 1)
    smat = (jnp.bitwise_and(s_row, 3) == s_col).astype(jnp.float32)
    sel = jnp.dot(qf, smat, precision=hi)
    pred = rois + sel
    px1, py1, px2, py2 = (pred[:, k:k + 1] for k in range(4))

    gx1, gy1, gx2, gy2 = (gtt[k:k + 1, :] for k in range(4))
    area_a = (px2 - px1) * (py2 - py1)
    area_b = (gx2 - gx1) * (gy2 - gy1)
    iw = jnp.maximum(jnp.minimum(px2, gx2) - jnp.maximum(px1, gx1), 0.0)
    ih = jnp.maximum(jnp.minimum(py2, gy2) - jnp.maximum(py1, gy1), 0.0)
    inter = iw * ih
    iou = inter / (area_a + area_b - inter + 1e-9)
    max_iou = jnp.max(iou, axis=1, keepdims=True)
    lane_m = jax.lax.broadcasted_iota(jnp.int32, (tl, M), 1)
    arg = jnp.min(jnp.where(iou == max_iou, lane_m, M), axis=1, keepdims=True)

    pos = max_iou >= _UPPER
    neg = max_iou < _LOWER
    onehot = (lane_m == arg).astype(jnp.float32)
    gsel = jnp.dot(onehot, g, precision=hi)
    pos_label = gsel[:, 4:5]
    label = jnp.where(pos, pos_label, float(_BACKGROUND))

    logit_at = jnp.sum(
        jnp.where(lane_c.astype(jnp.float32) == label, cls, 0.0),
        axis=1, keepdims=True)
    ce = logz - logit_at
    w = (pos | neg).astype(jnp.float32)

    d4 = pred - gsel[:, 0:4]
    ad = jnp.abs(d4)
    bl4 = jnp.where(ad < 1.0, 0.5 * d4 * d4, ad - 0.5)
    bl = jnp.sum(bl4, axis=1, keepdims=True)
    pw = pos.astype(jnp.float32)

    sums = (jnp.sum(ce * w), jnp.sum(w), jnp.sum(pw),
            jnp.sum(neg.astype(jnp.float32)), jnp.sum(bl * pw))
    lane_o = jax.lax.broadcasted_iota(jnp.int32, (1, 128), 1)
    vec = jnp.zeros((1, 128), jnp.float32)
    for j, sv in enumerate(sums):
        vec = vec + jnp.where(lane_o == j, sv, 0.0)

    @pl.when(t == 0)
    def _init():
        acc_ref[0] = vec

    @pl.when(t != 0)
    def _acc():
        acc_ref[0] = acc_ref[0] + vec


def kernel(rois, cls_scores, bbox_deltas, gt_boxes, gt_clses, device):
    N, L, C = cls_scores.shape
    M = gt_boxes.shape[2]
    gtt = jnp.swapaxes(gt_boxes[:, 0], 1, 2)
    g = jnp.concatenate(
        [gt_boxes[:, 0], gt_clses.astype(jnp.float32)[:, :, None],
         jnp.zeros((N, M, 3), jnp.float32)], axis=-1)
    T = L // _TL
    acc = pl.pallas_call(
        _loss_kernel,
        grid=(N, T),
        in_specs=[
            pl.BlockSpec((1, _TL, C), lambda n, t: (n, t, 0)),
            pl.BlockSpec((1, _TL, 4 * C), lambda n, t: (n, t, 0)),
            pl.BlockSpec((1, _TL, 4), lambda n, t: (n, t, 0)),
            pl.BlockSpec((1, 4, M), lambda n, t: (n, 0, 0)),
            pl.BlockSpec((1, M, 8), lambda n, t: (n, 0, 0)),
        ],
        out_specs=pl.BlockSpec((1, 1, 128), lambda n, t: (n, 0, 0)),
        out_shape=jax.ShapeDtypeStruct((N, 1, 128), jnp.float32),
    )(cls_scores, bbox_deltas, rois, gtt, g)
    acc = acc[:, 0, :]
    s_ce_w, s_w, s_pos, s_neg, s_bl = (acc[:, j] for j in range(5))
    cls_loss = jnp.sum(s_ce_w / jnp.maximum(s_w, 1.0))
    bbox_loss = jnp.sum(jnp.where(s_pos > 0, s_bl / N, 0.0))
    return (cls_loss, bbox_loss, jnp.sum(s_pos), jnp.sum(s_neg))


# speedup vs baseline: 1.5547x; 1.5547x over previous
import jax
import jax.numpy as jnp
from jax.experimental import pallas as pl

_NEGATIVE = -2
_UPPER = 0.4
_LOWER = 0.1
_NCLS = 80
_BACKGROUND = _NCLS
_TL = 1000


def _loss_kernel(cls_ref, bd_ref, rois_ref, gtt_ref, gtc_ref, acc_ref):
    t = pl.program_id(1)
    cls = cls_ref[0]
    bd = bd_ref[0]
    rois = rois_ref[0]
    gtt = gtt_ref[0]
    gtc = gtc_ref[0]

    tl, C = cls.shape
    M = gtc.shape[1]

    lane_c = jax.lax.broadcasted_iota(jnp.int32, (tl, C), 1)
    idx = jnp.argmax(cls, axis=1, keepdims=True).astype(jnp.int32)

    logz = jnp.log(jnp.sum(jnp.exp(cls), axis=1, keepdims=True))

    D = bd.shape[1]
    lane_d = jax.lax.broadcasted_iota(jnp.int32, (tl, D), 1)
    cls_hit = jax.lax.shift_right_logical(lane_d, 2) == idx
    q = jnp.where(cls_hit, bd, 0.0)
    tail = jnp.concatenate(
        [q[:, 256:D], jnp.zeros((tl, 384 - D), jnp.float32)], axis=1)
    qf = q[:, 0:128] + q[:, 128:256] + tail
    lane_f = jax.lax.broadcasted_iota(jnp.int32, (tl, 128), 1)
    sub = jnp.bitwise_and(lane_f, 3)
    pred = []
    for k in range(4):
        sk = jnp.sum(jnp.where(sub == k, qf, 0.0), axis=1, keepdims=True)
        pred.append(rois[:, k:k + 1] + sk)
    px1, py1, px2, py2 = pred

    gx1, gy1, gx2, gy2 = (gtt[k:k + 1, :] for k in range(4))
    area_a = (px2 - px1) * (py2 - py1)
    area_b = (gx2 - gx1) * (gy2 - gy1)
    iw = jnp.maximum(jnp.minimum(px2, gx2) - jnp.maximum(px1, gx1), 0.0)
    ih = jnp.maximum(jnp.minimum(py2, gy2) - jnp.maximum(py1, gy1), 0.0)
    inter = iw * ih
    iou = inter / (area_a + area_b - inter + 1e-9)
    max_iou = jnp.max(iou, axis=1, keepdims=True)
    lane_m = jax.lax.broadcasted_iota(jnp.int32, (tl, M), 1)
    arg = jnp.argmax(iou, axis=1, keepdims=True).astype(jnp.int32)

    pos = max_iou >= _UPPER
    neg = max_iou < _LOWER
    onehot = lane_m == arg
    pos_label = jnp.sum(jnp.where(onehot, gtc, 0.0), axis=1, keepdims=True)
    label = jnp.where(pos, pos_label, float(_BACKGROUND))

    logit_at = jnp.sum(
        jnp.where(lane_c.astype(jnp.float32) == label, cls, 0.0),
        axis=1, keepdims=True)
    ce = logz - logit_at
    w = (pos | neg).astype(jnp.float32)

    ds = []
    for k in range(4):
        gk = jnp.sum(jnp.where(onehot, gtt[k:k + 1, :], 0.0), axis=1, keepdims=True)
        ds.append(pred[k] - gk)
    d4 = jnp.concatenate(ds, axis=1)
    ad = jnp.abs(d4)
    bl4 = jnp.where(ad < 1.0, 0.5 * d4 * d4, ad - 0.5)
    bl = jnp.sum(bl4, axis=1, keepdims=True)
    pw = pos.astype(jnp.float32)

    sums = (jnp.sum(ce * w), jnp.sum(w), jnp.sum(pw),
            jnp.sum(neg.astype(jnp.float32)), jnp.sum(bl * pw))
    lane_o = jax.lax.broadcasted_iota(jnp.int32, (1, 128), 1)
    vec = jnp.zeros((1, 128), jnp.float32)
    for j, sv in enumerate(sums):
        vec = vec + jnp.where(lane_o == j, sv, 0.0)

    @pl.when(t == 0)
    def _init():
        acc_ref[0] = vec

    @pl.when(t != 0)
    def _acc():
        acc_ref[0] = acc_ref[0] + vec


def kernel(rois, cls_scores, bbox_deltas, gt_boxes, gt_clses, device):
    N, L, C = cls_scores.shape
    M = gt_boxes.shape[2]
    gtt = jnp.swapaxes(gt_boxes[:, 0], 1, 2)
    gtc = gt_clses.astype(jnp.float32).reshape(N, 1, M)
    T = L // _TL
    acc = pl.pallas_call(
        _loss_kernel,
        grid=(N, T),
        in_specs=[
            pl.BlockSpec((1, _TL, C), lambda n, t: (n, t, 0)),
            pl.BlockSpec((1, _TL, 4 * C), lambda n, t: (n, t, 0)),
            pl.BlockSpec((1, _TL, 4), lambda n, t: (n, t, 0)),
            pl.BlockSpec((1, 4, M), lambda n, t: (n, 0, 0)),
            pl.BlockSpec((1, 1, M), lambda n, t: (n, 0, 0)),
        ],
        out_specs=pl.BlockSpec((1, 1, 128), lambda n, t: (n, 0, 0)),
        out_shape=jax.ShapeDtypeStruct((N, 1, 128), jnp.float32),
    )(cls_scores, bbox_deltas, rois, gtt, gtc)
    acc = acc[:, 0, :]
    s_ce_w, s_w, s_pos, s_neg, s_bl = (acc[:, j] for j in range(5))
    cls_loss = jnp.sum(s_ce_w / jnp.maximum(s_w, 1.0))
    bbox_loss = jnp.sum(jnp.where(s_pos > 0, s_bl / N, 0.0))
    return (cls_loss, bbox_loss, jnp.sum(s_pos), jnp.sum(s_neg))


# speedup vs baseline: 1.9821x; 1.2749x over previous
import jax
import jax.numpy as jnp
from jax.experimental import pallas as pl

_UPPER = 0.4
_LOWER = 0.1
_NCLS = 80
_BACKGROUND = _NCLS
_TL = 1000


def _loss_kernel(cls_ref, bd_ref, roist_ref, gtb_ref, acc_ref):
    t = pl.program_id(1)
    cls = cls_ref[0]
    bd = bd_ref[0]
    roist = roist_ref[0, 0]
    gtb = gtb_ref[0]

    tl, C = cls.shape
    M = gtb.shape[0]

    lane_c = jax.lax.broadcasted_iota(jnp.int32, (tl, C), 1)
    idx = jnp.argmax(cls, axis=1, keepdims=True).astype(jnp.int32)

    logz = jnp.log(jnp.sum(jnp.exp(cls), axis=1, keepdims=True))

    D = bd.shape[1]
    lane_d = jax.lax.broadcasted_iota(jnp.int32, (tl, D), 1)
    cls_hit = jax.lax.shift_right_logical(lane_d, 2) == idx
    q = jnp.where(cls_hit, bd, 0.0)
    tail = jnp.concatenate(
        [q[:, 256:D], jnp.zeros((tl, 384 - D), jnp.float32)], axis=1)
    qf = q[:, 0:128] + q[:, 128:256] + tail
    lane_f = jax.lax.broadcasted_iota(jnp.int32, (tl, 128), 1)
    sub = jnp.bitwise_and(lane_f, 3)
    sks = [jnp.sum(jnp.where(sub == k, qf, 0.0), axis=1, keepdims=True)
           for k in range(4)]

    pack = jnp.concatenate(
        sks + [logz, jnp.zeros((tl, 3), jnp.float32)], axis=1)
    packt = jnp.swapaxes(pack, 0, 1)
    predt = roist + packt[0:4, :]
    px1, py1, px2, py2 = (predt[k:k + 1, :] for k in range(4))
    logzr = packt[4:5, :]

    gx1, gy1, gx2, gy2 = (gtb[:, k:k + 1] for k in range(4))
    gtcf = gtb[:, 4:5]
    area_a = (px2 - px1) * (py2 - py1)
    area_b = (gx2 - gx1) * (gy2 - gy1)
    iw = jnp.maximum(jnp.minimum(px2, gx2) - jnp.maximum(px1, gx1), 0.0)
    ih = jnp.maximum(jnp.minimum(py2, gy2) - jnp.maximum(py1, gy1), 0.0)
    inter = iw * ih
    iou = inter / (area_a + area_b - inter + 1e-9)
    max_iou = jnp.max(iou, axis=0, keepdims=True)
    arg = jnp.argmax(iou, axis=0, keepdims=True).astype(jnp.int32)

    pos = max_iou >= _UPPER
    neg = max_iou < _LOWER
    sub_m = jax.lax.broadcasted_iota(jnp.int32, (M, 1), 0)
    onehot = sub_m == arg
    pos_label = jnp.sum(jnp.where(onehot, gtcf, 0.0), axis=0, keepdims=True)
    label = jnp.where(pos, pos_label, float(_BACKGROUND))

    labelt = jnp.swapaxes(label, 0, 1)
    logit_nat = jnp.sum(
        jnp.where(lane_c.astype(jnp.float32) == labelt, cls, 0.0),
        axis=1, keepdims=True)
    logit_at = jnp.swapaxes(logit_nat, 0, 1)
    ce = logzr - logit_at
    w = (pos | neg).astype(jnp.float32)

    gk = jnp.concatenate(
        [jnp.sum(jnp.where(onehot, gtb[:, k:k + 1], 0.0), axis=0,
                 keepdims=True) for k in range(4)], axis=0)
    d4 = predt - gk
    ad = jnp.abs(d4)
    bl4 = jnp.where(ad < 1.0, 0.5 * d4 * d4, ad - 0.5)
    bl = jnp.sum(bl4, axis=0, keepdims=True)
    pw = pos.astype(jnp.float32)

    sums = (jnp.sum(ce * w), jnp.sum(w), jnp.sum(pw),
            jnp.sum(neg.astype(jnp.float32)), jnp.sum(bl * pw))
    lane_o = jax.lax.broadcasted_iota(jnp.int32, (1, 128), 1)
    vec = jnp.zeros((1, 128), jnp.float32)
    for j, sv in enumerate(sums):
        vec = vec + jnp.where(lane_o == j, sv, 0.0)

    @pl.when(t == 0)
    def _init():
        acc_ref[0] = vec

    @pl.when(t != 0)
    def _acc():
        acc_ref[0] = acc_ref[0] + vec


def kernel(rois, cls_scores, bbox_deltas, gt_boxes, gt_clses, device):
    N, L, C = cls_scores.shape
    M = gt_boxes.shape[2]
    T = L // _TL
    roist = jnp.swapaxes(rois, 1, 2).reshape(
        N, 4, T, _TL).transpose(0, 2, 1, 3)
    gtb = jnp.concatenate(
        [gt_boxes[:, 0], gt_clses.astype(jnp.float32)[:, :, None],
         jnp.zeros((N, M, 3), jnp.float32)], axis=-1)
    acc = pl.pallas_call(
        _loss_kernel,
        grid=(N, T),
        in_specs=[
            pl.BlockSpec((1, _TL, C), lambda n, t: (n, t, 0)),
            pl.BlockSpec((1, _TL, 4 * C), lambda n, t: (n, t, 0)),
            pl.BlockSpec((1, 1, 4, _TL), lambda n, t: (n, t, 0, 0)),
            pl.BlockSpec((1, M, 8), lambda n, t: (n, 0, 0)),
        ],
        out_specs=pl.BlockSpec((1, 1, 128), lambda n, t: (n, 0, 0)),
        out_shape=jax.ShapeDtypeStruct((N, 1, 128), jnp.float32),
    )(cls_scores, bbox_deltas, roist, gtb)
    acc = acc[:, 0, :]
    s_ce_w, s_w, s_pos, s_neg, s_bl = (acc[:, j] for j in range(5))
    cls_loss = jnp.sum(s_ce_w / jnp.maximum(s_w, 1.0))
    bbox_loss = jnp.sum(jnp.where(s_pos > 0, s_bl / N, 0.0))
    return (cls_loss, bbox_loss, jnp.sum(s_pos), jnp.sum(s_neg))


# speedup vs baseline: 2.1108x; 1.0649x over previous
import jax
import jax.numpy as jnp
from jax.experimental import pallas as pl

_UPPER = 0.4
_LOWER = 0.1
_NCLS = 80
_BACKGROUND = _NCLS
_TL = 5000


def _loss_kernel(cls_ref, bd_ref, roist_ref, gtb_ref, acc_ref):
    t = pl.program_id(1)
    cls = cls_ref[0]
    bd = bd_ref[0]
    roist = roist_ref[0, 0]
    gtb = gtb_ref[0]

    tl, C = cls.shape
    M = gtb.shape[0]

    lane_c = jax.lax.broadcasted_iota(jnp.int32, (tl, C), 1)
    idx = jnp.argmax(cls, axis=1, keepdims=True).astype(jnp.int32)

    sumexp = jnp.sum(jnp.exp(cls), axis=1, keepdims=True)

    D = bd.shape[1]
    lane_d = jax.lax.broadcasted_iota(jnp.int32, (tl, D), 1)
    cls_hit = jax.lax.shift_right_logical(lane_d, 2) == idx
    q = jnp.where(cls_hit, bd, 0.0)
    tail = jnp.concatenate(
        [q[:, 256:D], jnp.zeros((tl, 384 - D), jnp.float32)], axis=1)
    qf = q[:, 0:128] + q[:, 128:256] + tail
    lane_f = jax.lax.broadcasted_iota(jnp.int32, (tl, 128), 1)
    sub = jnp.bitwise_and(lane_f, 3)
    sks = [jnp.sum(jnp.where(sub == k, qf, 0.0), axis=1, keepdims=True)
           for k in range(4)]

    pack = jnp.concatenate(
        sks + [sumexp, jnp.zeros((tl, 3), jnp.float32)], axis=1)
    packt = jnp.swapaxes(pack, 0, 1)
    predt = roist + packt[0:4, :]
    px1, py1, px2, py2 = (predt[k:k + 1, :] for k in range(4))
    logzr = jnp.log(packt[4:5, :])

    gx1, gy1, gx2, gy2 = (gtb[:, k:k + 1] for k in range(4))
    gtcf = gtb[:, 4:5]
    area_a = (px2 - px1) * (py2 - py1)
    area_b = (gx2 - gx1) * (gy2 - gy1)
    iw = jnp.maximum(jnp.minimum(px2, gx2) - jnp.maximum(px1, gx1), 0.0)
    ih = jnp.maximum(jnp.minimum(py2, gy2) - jnp.maximum(py1, gy1), 0.0)
    inter = iw * ih
    iou = inter / (area_a + area_b - inter + 1e-9)
    max_iou = jnp.max(iou, axis=0, keepdims=True)
    arg = jnp.argmax(iou, axis=0, keepdims=True).astype(jnp.int32)

    pos = max_iou >= _UPPER
    neg = max_iou < _LOWER
    sub_m = jax.lax.broadcasted_iota(jnp.int32, (M, 1), 0)
    onehot = sub_m == arg
    pos_label = jnp.sum(jnp.where(onehot, gtcf, 0.0), axis=0, keepdims=True)
    label = jnp.where(pos, pos_label, float(_BACKGROUND))

    labelt = jnp.swapaxes(label, 0, 1)
    logit_nat = jnp.sum(
        jnp.where(lane_c.astype(jnp.float32) == labelt, cls, 0.0),
        axis=1, keepdims=True)
    logit_at = jnp.swapaxes(logit_nat, 0, 1)
    ce = logzr - logit_at
    w = (pos | neg).astype(jnp.float32)

    gk = jnp.concatenate(
        [jnp.sum(jnp.where(onehot, gtb[:, k:k + 1], 0.0), axis=0,
                 keepdims=True) for k in range(4)], axis=0)
    d4 = predt - gk
    ad = jnp.abs(d4)
    bl4 = jnp.where(ad < 1.0, 0.5 * d4 * d4, ad - 0.5)
    bl = jnp.sum(bl4, axis=0, keepdims=True)
    pw = pos.astype(jnp.float32)

    sums = (jnp.sum(ce * w), jnp.sum(w), jnp.sum(pw),
            jnp.sum(neg.astype(jnp.float32)), jnp.sum(bl * pw))
    lane_o = jax.lax.broadcasted_iota(jnp.int32, (1, 128), 1)
    vec = jnp.zeros((1, 128), jnp.float32)
    for j, sv in enumerate(sums):
        vec = vec + jnp.where(lane_o == j, sv, 0.0)

    @pl.when(t == 0)
    def _init():
        acc_ref[0] = vec

    @pl.when(t != 0)
    def _acc():
        acc_ref[0] = acc_ref[0] + vec


def kernel(rois, cls_scores, bbox_deltas, gt_boxes, gt_clses, device):
    N, L, C = cls_scores.shape
    M = gt_boxes.shape[2]
    T = L // _TL
    roist = jnp.swapaxes(rois, 1, 2).reshape(
        N, 4, T, _TL).transpose(0, 2, 1, 3)
    gtb = jnp.concatenate(
        [gt_boxes[:, 0], gt_clses.astype(jnp.float32)[:, :, None],
         jnp.zeros((N, M, 3), jnp.float32)], axis=-1)
    acc = pl.pallas_call(
        _loss_kernel,
        grid=(N, T),
        in_specs=[
            pl.BlockSpec((1, _TL, C), lambda n, t: (n, t, 0)),
            pl.BlockSpec((1, _TL, 4 * C), lambda n, t: (n, t, 0)),
            pl.BlockSpec((1, 1, 4, _TL), lambda n, t: (n, t, 0, 0)),
            pl.BlockSpec((1, M, 8), lambda n, t: (n, 0, 0)),
        ],
        out_specs=pl.BlockSpec((1, 1, 128), lambda n, t: (n, 0, 0)),
        out_shape=jax.ShapeDtypeStruct((N, 1, 128), jnp.float32),
    )(cls_scores, bbox_deltas, roist, gtb)
    acc = acc[:, 0, :]
    s_ce_w, s_w, s_pos, s_neg, s_bl = (acc[:, j] for j in range(5))
    cls_loss = jnp.sum(s_ce_w / jnp.maximum(s_w, 1.0))
    bbox_loss = jnp.sum(jnp.where(s_pos > 0, s_bl / N, 0.0))
    return (cls_loss, bbox_loss, jnp.sum(s_pos), jnp.sum(s_neg))


# speedup vs baseline: 2.1213x; 1.0050x over previous
import jax
import jax.numpy as jnp
from jax.experimental import pallas as pl

_UPPER = 0.4
_LOWER = 0.1
_NCLS = 80
_BACKGROUND = _NCLS
_TL = 5000


def _loss_kernel(cls_ref, bd_ref, roist_ref, gtb_ref, acc_ref):
    t = pl.program_id(1)
    cls = cls_ref[0]
    bd = bd_ref[0]
    roist = roist_ref[0, 0]
    gtb = gtb_ref[0]

    tl, C = cls.shape
    M = gtb.shape[0]

    lane_c = jax.lax.broadcasted_iota(jnp.int32, (tl, C), 1)
    idx = jnp.argmax(cls, axis=1, keepdims=True).astype(jnp.int32)

    sumexp = jnp.sum(jnp.exp(cls), axis=1, keepdims=True)

    D = bd.shape[1]
    lane_d = jax.lax.broadcasted_iota(jnp.int32, (tl, D), 1)
    cls_hit = jax.lax.shift_right_logical(lane_d, 2) == idx
    q = jnp.where(cls_hit, bd, 0.0)
    tail = jnp.concatenate(
        [q[:, 256:D], jnp.zeros((tl, 384 - D), jnp.float32)], axis=1)
    qf = q[:, 0:128] + q[:, 128:256] + tail
    lane_f = jax.lax.broadcasted_iota(jnp.int32, (tl, 128), 1)
    sub = jnp.bitwise_and(lane_f, 3)
    sks = [jnp.sum(jnp.where(sub == k, qf, 0.0), axis=1, keepdims=True)
           for k in range(4)]

    pack = jnp.concatenate(
        sks + [sumexp, jnp.zeros((tl, 3), jnp.float32)], axis=1)
    packt = jnp.swapaxes(pack, 0, 1)
    predt = roist + packt[0:4, :]
    px1, py1, px2, py2 = (predt[k:k + 1, :] for k in range(4))
    logzr = jnp.log(packt[4:5, :])

    gx1, gy1, gx2, gy2 = (gtb[:, k:k + 1] for k in range(4))
    gtcf = gtb[:, 4:5]
    area_a = (px2 - px1) * (py2 - py1)
    area_b = (gx2 - gx1) * (gy2 - gy1)
    iw = jnp.maximum(jnp.minimum(px2, gx2) - jnp.maximum(px1, gx1), 0.0)
    ih = jnp.maximum(jnp.minimum(py2, gy2) - jnp.maximum(py1, gy1), 0.0)
    inter = iw * ih
    iou = inter / (area_a + area_b - inter + 1e-9)
    max_iou = jnp.max(iou, axis=0, keepdims=True)
    arg = jnp.argmax(iou, axis=0, keepdims=True).astype(jnp.int32)

    pos = max_iou >= _UPPER
    neg = max_iou < _LOWER
    sub_m = jax.lax.broadcasted_iota(jnp.int32, (M, 1), 0)
    onehot = sub_m == arg
    pos_label = jnp.sum(jnp.where(onehot, gtcf, 0.0), axis=0, keepdims=True)
    label = jnp.where(pos, pos_label, float(_BACKGROUND))

    labelt = jnp.swapaxes(label, 0, 1)
    logit_nat = jnp.sum(
        jnp.where(lane_c.astype(jnp.float32) == labelt, cls, 0.0),
        axis=1, keepdims=True)
    logit_at = jnp.swapaxes(logit_nat, 0, 1)
    ce = logzr - logit_at
    w = (pos | neg).astype(jnp.float32)

    gk = jnp.concatenate(
        [jnp.sum(jnp.where(onehot, gtb[:, k:k + 1], 0.0), axis=0,
                 keepdims=True) for k in range(4)], axis=0)
    d4 = predt - gk
    ad = jnp.abs(d4)
    bl4 = jnp.where(ad < 1.0, 0.5 * d4 * d4, ad - 0.5)
    bl = jnp.sum(bl4, axis=0, keepdims=True)
    pw = pos.astype(jnp.float32)

    sums = (jnp.sum(ce * w), jnp.sum(pw),
            jnp.sum(neg.astype(jnp.float32)), jnp.sum(bl * pw))
    lane_o = jax.lax.broadcasted_iota(jnp.int32, (1, 128), 1)
    vec = jnp.zeros((1, 128), jnp.float32)
    for j, sv in enumerate(sums):
        vec = vec + jnp.where(lane_o == j, sv, 0.0)

    @pl.when(t == 0)
    def _init():
        acc_ref[0] = vec

    @pl.when(t != 0)
    def _acc():
        acc_ref[0] = acc_ref[0] + vec


def kernel(rois, cls_scores, bbox_deltas, gt_boxes, gt_clses, device):
    N, L, C = cls_scores.shape
    M = gt_boxes.shape[2]
    T = L // _TL
    roist = jnp.swapaxes(rois, 1, 2).reshape(
        N, 4, T, _TL).transpose(0, 2, 1, 3)
    gtb = jnp.concatenate(
        [gt_boxes[:, 0], gt_clses.astype(jnp.float32)[:, :, None],
         jnp.zeros((N, M, 3), jnp.float32)], axis=-1)
    acc = pl.pallas_call(
        _loss_kernel,
        grid=(N, T),
        in_specs=[
            pl.BlockSpec((1, _TL, C), lambda n, t: (n, t, 0)),
            pl.BlockSpec((1, _TL, 4 * C), lambda n, t: (n, t, 0)),
            pl.BlockSpec((1, 1, 4, _TL), lambda n, t: (n, t, 0, 0)),
            pl.BlockSpec((1, M, 8), lambda n, t: (n, 0, 0)),
        ],
        out_specs=pl.BlockSpec((1, 1, 128), lambda n, t: (n, 0, 0)),
        out_shape=jax.ShapeDtypeStruct((N, 1, 128), jnp.float32),
    )(cls_scores, bbox_deltas, roist, gtb)
    acc = acc[:, 0, :]
    s_ce_w, s_pos, s_neg, s_bl = (acc[:, j] for j in range(4))
    cls_loss = jnp.sum(s_ce_w / jnp.maximum(s_pos + s_neg, 1.0))
    bbox_loss = jnp.sum(jnp.where(s_pos > 0, s_bl / N, 0.0))
    return (cls_loss, bbox_loss, jnp.sum(s_pos), jnp.sum(s_neg))


# speedup vs baseline: 4.5414x; 2.1409x over previous
import jax
import jax.numpy as jnp
from jax import lax
from jax.experimental import pallas as pl

_UPPER = 0.4
_LOWER = 0.1
_NCLS = 80
_BACKGROUND = _NCLS
_TL = 640


def _loss_kernel(clst_ref, bdt_ref, roist_ref, gtbt_ref, acc_ref):
    t = pl.program_id(0)
    clst = clst_ref[...]
    bdt = bdt_ref[...]
    roist = roist_ref[...]
    gtbt = gtbt_ref[...]

    C, N, tl = clst.shape
    M = gtbt.shape[0]

    idx = jnp.argmax(clst, axis=0).astype(jnp.int32)
    logz = jnp.log(jnp.sum(jnp.exp(clst), axis=0))

    ci = lax.broadcasted_iota(jnp.int32, (4 * C, 1, 1), 0)
    hit = lax.shift_right_logical(ci, 2) == idx[None]
    q = jnp.where(hit, bdt, 0.0)
    sel = jnp.sum(q.reshape(C, 4, N, tl), axis=0)
    pred = roist + sel
    px1, py1, px2, py2 = (pred[k:k + 1] for k in range(4))

    gx1, gy1, gx2, gy2 = (gtbt[:, :, k:k + 1] for k in range(4))
    gtcf = gtbt[:, :, 4:5]
    area_a = (px2 - px1) * (py2 - py1)
    area_b = (gx2 - gx1) * (gy2 - gy1)
    iw = jnp.maximum(jnp.minimum(px2, gx2) - jnp.maximum(px1, gx1), 0.0)
    ih = jnp.maximum(jnp.minimum(py2, gy2) - jnp.maximum(py1, gy1), 0.0)
    inter = iw * ih
    iou = inter / (area_a + area_b - inter + 1e-9)
    max_iou = jnp.max(iou, axis=0)
    arg = jnp.argmax(iou, axis=0).astype(jnp.int32)

    pos = max_iou >= _UPPER
    neg = max_iou < _LOWER
    mi = lax.broadcasted_iota(jnp.int32, (M, 1, 1), 0)
    onehot = mi == arg[None]
    pos_label = jnp.sum(jnp.where(onehot, gtcf, 0.0), axis=0)
    label = jnp.where(pos, pos_label, float(_BACKGROUND))

    cc = lax.broadcasted_iota(jnp.int32, (C, 1, 1), 0)
    lhit = cc.astype(jnp.float32) == label[None]
    logit_at = jnp.sum(jnp.where(lhit, clst, 0.0), axis=0)
    ce = logz - logit_at

    bl = jnp.zeros((N, tl), jnp.float32)
    for k in range(4):
        gk = jnp.sum(jnp.where(onehot, gtbt[:, :, k:k + 1], 0.0), axis=0)
        d = pred[k] - gk
        ad = jnp.abs(d)
        bl = bl + jnp.where(ad < 1.0, 0.5 * d * d, ad - 0.5)

    lane = lax.broadcasted_iota(jnp.int32, (N, tl), 1)
    valid = lane < (5000 - t * tl)
    wm = (pos | neg) & valid
    pm = pos & valid
    nm = neg & valid
    parts = (jnp.sum(jnp.where(wm, ce, 0.0), axis=1, keepdims=True),
             jnp.sum(jnp.where(pm, 1.0, 0.0), axis=1, keepdims=True),
             jnp.sum(jnp.where(nm, 1.0, 0.0), axis=1, keepdims=True),
             jnp.sum(jnp.where(pm, bl, 0.0), axis=1, keepdims=True))
    lane_o = lax.broadcasted_iota(jnp.int32, (N, 128), 1)
    vec = jnp.zeros((N, 128), jnp.float32)
    for j, sv in enumerate(parts):
        vec = vec + jnp.where(lane_o == j, sv, 0.0)

    @pl.when(t == 0)
    def _init():
        acc_ref[0] = vec

    @pl.when(t != 0)
    def _acc():
        acc_ref[0] = acc_ref[0] + vec


def kernel(rois, cls_scores, bbox_deltas, gt_boxes, gt_clses, device):
    N, L, C = cls_scores.shape
    M = gt_boxes.shape[2]
    clst = jnp.transpose(cls_scores, (2, 0, 1))
    bdt = jnp.transpose(bbox_deltas, (2, 0, 1))
    roist = jnp.transpose(rois, (2, 0, 1))
    gtbt = jnp.concatenate(
        [jnp.transpose(gt_boxes[:, 0], (1, 0, 2)),
         jnp.transpose(gt_clses.astype(jnp.float32))[:, :, None],
         jnp.zeros((M, N, 3), jnp.float32)], axis=-1)
    T = -(-L // _TL)
    acc = pl.pallas_call(
        _loss_kernel,
        grid=(T,),
        in_specs=[
            pl.BlockSpec((C, N, _TL), lambda t: (0, 0, t)),
            pl.BlockSpec((4 * C, N, _TL), lambda t: (0, 0, t)),
            pl.BlockSpec((4, N, _TL), lambda t: (0, 0, t)),
            pl.BlockSpec((M, N, 8), lambda t: (0, 0, 0)),
        ],
        out_specs=pl.BlockSpec((1, N, 128), lambda t: (0, 0, 0)),
        out_shape=jax.ShapeDtypeStruct((1, N, 128), jnp.float32),
    )(clst, bdt, roist, gtbt)
    acc = acc[0]
    s_ce_w, s_pos, s_neg, s_bl = (acc[:, j] for j in range(4))
    cls_loss = jnp.sum(s_ce_w / jnp.maximum(s_pos + s_neg, 1.0))
    bbox_loss = jnp.sum(jnp.where(s_pos > 0, s_bl / N, 0.0))
    return (cls_loss, bbox_loss, jnp.sum(s_pos), jnp.sum(s_neg))


# speedup vs baseline: 4.5460x; 1.0010x over previous
import functools

import jax
import jax.numpy as jnp
from jax import lax
from jax.experimental import pallas as pl

_UPPER = 0.4
_LOWER = 0.1
_NCLS = 80
_BACKGROUND = _NCLS
_TL = 640


def _loss_kernel(clst_ref, bdt_ref, roist_ref, gtbt_ref, acc_ref, *, L):
    t = pl.program_id(0)
    clst = clst_ref[...]
    bdt = bdt_ref[...]
    roist = roist_ref[...]
    gtbt = gtbt_ref[...]

    C, N, tl = clst.shape
    M = gtbt.shape[0]

    idx = jnp.argmax(clst, axis=0).astype(jnp.int32)
    logz = jnp.log(jnp.sum(jnp.exp(clst), axis=0))

    ci = lax.broadcasted_iota(jnp.int32, (4 * C, 1, 1), 0)
    hit = lax.shift_right_logical(ci, 2) == idx[None]
    q = jnp.where(hit, bdt, 0.0)
    sel = jnp.sum(q.reshape(C, 4, N, tl), axis=0)
    pred = roist + sel
    px1, py1, px2, py2 = (pred[k:k + 1] for k in range(4))

    gx1, gy1, gx2, gy2 = (gtbt[:, :, k:k + 1] for k in range(4))
    gtcf = gtbt[:, :, 4:5]
    area_a = (px2 - px1) * (py2 - py1)
    area_b = (gx2 - gx1) * (gy2 - gy1)
    iw = jnp.maximum(jnp.minimum(px2, gx2) - jnp.maximum(px1, gx1), 0.0)
    ih = jnp.maximum(jnp.minimum(py2, gy2) - jnp.maximum(py1, gy1), 0.0)
    inter = iw * ih
    iou = inter / (area_a + area_b - inter + 1e-9)
    max_iou = jnp.max(iou, axis=0)
    arg = jnp.argmax(iou, axis=0).astype(jnp.int32)

    pos = max_iou >= _UPPER
    neg = max_iou < _LOWER
    mi = lax.broadcasted_iota(jnp.int32, (M, 1, 1), 0)
    onehot = mi == arg[None]
    pos_label = jnp.sum(jnp.where(onehot, gtcf, 0.0), axis=0)
    label = jnp.where(pos, pos_label, float(_BACKGROUND))

    cc = lax.broadcasted_iota(jnp.int32, (C, 1, 1), 0)
    lhit = cc.astype(jnp.float32) == label[None]
    logit_at = jnp.sum(jnp.where(lhit, clst, 0.0), axis=0)
    ce = logz - logit_at

    bl = jnp.zeros((N, tl), jnp.float32)
    for k in range(4):
        gk = jnp.sum(jnp.where(onehot, gtbt[:, :, k:k + 1], 0.0), axis=0)
        d = pred[k] - gk
        ad = jnp.abs(d)
        bl = bl + jnp.where(ad < 1.0, 0.5 * d * d, ad - 0.5)

    lane = lax.broadcasted_iota(jnp.int32, (N, tl), 1)
    valid = lane < (L - t * tl)
    wm = (pos | neg) & valid
    pm = pos & valid
    nm = neg & valid
    parts = (jnp.sum(jnp.where(wm, ce, 0.0), axis=1, keepdims=True),
             jnp.sum(jnp.where(pm, 1.0, 0.0), axis=1, keepdims=True),
             jnp.sum(jnp.where(nm, 1.0, 0.0), axis=1, keepdims=True),
             jnp.sum(jnp.where(pm, bl, 0.0), axis=1, keepdims=True))
    lane_o = lax.broadcasted_iota(jnp.int32, (N, 128), 1)
    vec = jnp.zeros((N, 128), jnp.float32)
    for j, sv in enumerate(parts):
        vec = vec + jnp.where(lane_o == j, sv, 0.0)

    @pl.when(t == 0)
    def _init():
        acc_ref[0] = vec

    @pl.when(t != 0)
    def _acc():
        acc_ref[0] = acc_ref[0] + vec


def kernel(rois, cls_scores, bbox_deltas, gt_boxes, gt_clses, device):
    N, L, C = cls_scores.shape
    M = gt_boxes.shape[2]
    clst = jnp.transpose(cls_scores, (2, 0, 1))
    bdt = jnp.transpose(bbox_deltas, (2, 0, 1))
    roist = jnp.transpose(rois, (2, 0, 1))
    gtbt = jnp.concatenate(
        [jnp.transpose(gt_boxes[:, 0], (1, 0, 2)),
         jnp.transpose(gt_clses.astype(jnp.float32))[:, :, None],
         jnp.zeros((M, N, 3), jnp.float32)], axis=-1)
    T = -(-L // _TL)
    acc = pl.pallas_call(
        functools.partial(_loss_kernel, L=L),
        grid=(T,),
        in_specs=[
            pl.BlockSpec((C, N, _TL), lambda t: (0, 0, t)),
            pl.BlockSpec((4 * C, N, _TL), lambda t: (0, 0, t)),
            pl.BlockSpec((4, N, _TL), lambda t: (0, 0, t)),
            pl.BlockSpec((M, N, 8), lambda t: (0, 0, 0)),
        ],
        out_specs=pl.BlockSpec((1, N, 128), lambda t: (0, 0, 0)),
        out_shape=jax.ShapeDtypeStruct((1, N, 128), jnp.float32),
    )(clst, bdt, roist, gtbt)
    acc = acc[0]
    s_ce_w, s_pos, s_neg, s_bl = (acc[:, j] for j in range(4))
    cls_loss = jnp.sum(s_ce_w / jnp.maximum(s_pos + s_neg, 1.0))
    bbox_loss = jnp.sum(jnp.where(s_pos > 0, s_bl / N, 0.0))
    return (cls_loss, bbox_loss, jnp.sum(s_pos), jnp.sum(s_neg))
